# Initial kernel scaffold; baseline (speedup 1.0000x reference)
import jax
import jax.numpy as jnp
from jax.experimental import pallas as pl


def kernel(h, x, edge_index, edge_attr, params):
    raise NotImplementedError("write your pallas kernel here")


# speedup vs baseline: 2.8032x; 2.8032x over previous
import functools

import jax
import jax.numpy as jnp
from jax import lax
from jax.experimental import pallas as pl
from jax.experimental.pallas import tpu as pltpu
from jax.experimental.pallas import tpu_sc as plsc

NN = 100000
NE = 1600000
HD = 32
HHALF = HD // 2
ED = 16
CW = 16

NC, NS = 2, 16
NW = NC * NS

G = 128

MROW_G = 5
CH_G = MROW_G * G
NCH_G = NE // CH_G
IT_G = -(-NCH_G // NW)

MROW_S = 4
CH_S = MROW_S * G
NCH_S = NE // CH_S
IT_S = -(-NCH_S // NS)
IT_C = -(-NCH_S // NW)

NNP = 102400
NEP = 1605632
STR = NNP // NS
SACC = NN // NS

BE = 2048
BN = 2048

_SC_MESH = plsc.VectorSubcoreMesh(
    core_axis_name="c", subcore_axis_name="s", num_cores=NC, num_subcores=NS)


def _gather_body(hh, cpt, row2, col2, hrow, hcol, crow, ccol,
                 ir0, ir1, ir2, ir3, ir4, ic0, ic1, ic2, ic3, ic4,
                 bhr, bhc, bcr, bcc, sem):
    c = lax.axis_index("c")
    s = lax.axis_index("s")
    w = s * NC + c
    irs = (ir0, ir1, ir2, ir3, ir4)
    ics = (ic0, ic1, ic2, ic3, ic4)

    def body(i, carry):
        k = i * NW + w

        @pl.when(k < NCH_G)
        def _():
            for j in range(MROW_G):
                pltpu.sync_copy(row2.at[k * MROW_G + j], irs[j])
                pltpu.sync_copy(col2.at[k * MROW_G + j], ics[j])
            cps = []
            for j in range(MROW_G):
                sl = pl.ds(j * G, G)
                cps.append(pltpu.async_copy(hh.at[irs[j]], bhr.at[sl], sem))
                cps.append(pltpu.async_copy(hh.at[ics[j]], bhc.at[sl], sem))
                cps.append(pltpu.async_copy(cpt.at[irs[j]], bcr.at[sl], sem))
                cps.append(pltpu.async_copy(cpt.at[ics[j]], bcc.at[sl], sem))
            for h_ in cps:
                h_.wait()
            off = k * CH_G
            pltpu.sync_copy(bhr, hrow.at[pl.ds(off, CH_G)])
            pltpu.sync_copy(bhc, hcol.at[pl.ds(off, CH_G)])
            pltpu.sync_copy(bcr, crow.at[pl.ds(off, CH_G)])
            pltpu.sync_copy(bcc, ccol.at[pl.ds(off, CH_G)])

        return carry

    lax.fori_loop(0, IT_G, body, 0)


_gather_call = pl.kernel(
    _gather_body,
    out_type=(
        jax.ShapeDtypeStruct((NEP, HD), jnp.float32),
        jax.ShapeDtypeStruct((NEP, HD), jnp.float32),
        jax.ShapeDtypeStruct((NEP, CW), jnp.float32),
        jax.ShapeDtypeStruct((NEP, CW), jnp.float32),
    ),
    mesh=_SC_MESH,
    compiler_params=pltpu.CompilerParams(use_tc_tiling_on_sc=False),
    scratch_types=(
        *([pltpu.VMEM((G,), jnp.int32)] * (2 * MROW_G)),
        pltpu.VMEM((CH_G, HD), jnp.float32),
        pltpu.VMEM((CH_G, HD), jnp.float32),
        pltpu.VMEM((CH_G, CW), jnp.float32),
        pltpu.VMEM((CH_G, CW), jnp.float32),
        pltpu.SemaphoreType.DMA,
    ),
)


def _scatter_body(m2a, m2b, tx, ty, tz, row2, z16, z1,
                  aggA, aggB, ax, ay, az,
                  ib0, ib1, ib2, ib3, mb, txb, tyb, tzb,
                  accM, accX, accY, accZ, sem):
    c = lax.axis_index("c")
    s = lax.axis_index("s")
    ibs = (ib0, ib1, ib2, ib3)

    pltpu.sync_copy(z16, accM.at[pl.ds(s * SACC, SACC)])

    @pl.when(c == 0)
    def _():
        pltpu.sync_copy(z1, accX.at[pl.ds(s * STR, STR)])
        pltpu.sync_copy(z1, accY.at[pl.ds(s * STR, STR)])
        pltpu.sync_copy(z1, accZ.at[pl.ds(s * STR, STR)])

    plsc.subcore_barrier()

    def body(i, carry):
        k = i * NS + s

        @pl.when(k < NCH_S)
        def _():
            off = k * CH_S
            for j in range(MROW_S):
                pltpu.sync_copy(row2.at[k * MROW_S + j], ibs[j])

            @pl.when(c == 0)
            def _():
                pltpu.sync_copy(m2a.at[pl.ds(off, CH_S)], mb)
                pltpu.sync_copy(tx.at[pl.ds(off, CH_S)], txb)
                pltpu.sync_copy(ty.at[pl.ds(off, CH_S)], tyb)
                pltpu.sync_copy(tz.at[pl.ds(off, CH_S)], tzb)

            @pl.when(c == 1)
            def _():
                pltpu.sync_copy(m2b.at[pl.ds(off, CH_S)], mb)

            cps = []
            for j in range(MROW_S):
                sl = pl.ds(j * G, G)
                cps.append(pltpu.async_copy(
                    mb.at[sl], accM.at[ibs[j]], sem, add=True))
            for h_ in cps:
                h_.wait()

            @pl.when(c == 0)
            def _():
                cps2 = []
                for j in range(MROW_S):
                    sl = pl.ds(j * G, G)
                    cps2.append(pltpu.async_copy(
                        txb.at[sl], accX.at[ibs[j]], sem, add=True))
                    cps2.append(pltpu.async_copy(
                        tyb.at[sl], accY.at[ibs[j]], sem, add=True))
                    cps2.append(pltpu.async_copy(
                        tzb.at[sl], accZ.at[ibs[j]], sem, add=True))
                for h_ in cps2:
                    h_.wait()

        return carry

    lax.fori_loop(0, IT_S, body, 0)
    plsc.subcore_barrier()

    @pl.when(c == 0)
    def _():
        pltpu.sync_copy(accM.at[pl.ds(s * SACC, SACC)],
                        aggA.at[pl.ds(s * SACC, SACC)])
        pltpu.sync_copy(accX.at[pl.ds(s * STR, STR)], ax.at[pl.ds(s * STR, STR)])
        pltpu.sync_copy(accY.at[pl.ds(s * STR, STR)], ay.at[pl.ds(s * STR, STR)])
        pltpu.sync_copy(accZ.at[pl.ds(s * STR, STR)], az.at[pl.ds(s * STR, STR)])

    @pl.when(c == 1)
    def _():
        pltpu.sync_copy(accM.at[pl.ds(s * SACC, SACC)],
                        aggB.at[pl.ds(s * SACC, SACC)])


_scatter_call = pl.kernel(
    _scatter_body,
    out_type=(
        jax.ShapeDtypeStruct((NNP, HHALF), jnp.float32),
        jax.ShapeDtypeStruct((NNP, HHALF), jnp.float32),
        jax.ShapeDtypeStruct((NNP,), jnp.float32),
        jax.ShapeDtypeStruct((NNP,), jnp.float32),
        jax.ShapeDtypeStruct((NNP,), jnp.float32),
    ),
    mesh=_SC_MESH,
    compiler_params=pltpu.CompilerParams(use_tc_tiling_on_sc=False),
    scratch_types=(
        *([pltpu.VMEM((G,), jnp.int32)] * MROW_S),
        pltpu.VMEM((CH_S, HHALF), jnp.float32),
        pltpu.VMEM((CH_S,), jnp.float32),
        pltpu.VMEM((CH_S,), jnp.float32),
        pltpu.VMEM((CH_S,), jnp.float32),
        pltpu.VMEM_SHARED((NN, HHALF), jnp.float32),
        pltpu.VMEM_SHARED((NNP,), jnp.float32),
        pltpu.VMEM_SHARED((NNP,), jnp.float32),
        pltpu.VMEM_SHARED((NNP,), jnp.float32),
        pltpu.SemaphoreType.DMA,
    ),
)


def _count_body(row2, ones, z1, cnt0, cnt1, ib0, ib1, ib2, ib3, ob, acc1, sem):
    c = lax.axis_index("c")
    s = lax.axis_index("s")
    w = s * NC + c
    ibs = (ib0, ib1, ib2, ib3)

    pltpu.sync_copy(z1, acc1.at[pl.ds(s * STR, STR)])
    pltpu.sync_copy(ones, ob)
    plsc.subcore_barrier()

    def body(i, carry):
        k = i * NW + w

        @pl.when(k < NCH_S)
        def _():
            for j in range(MROW_S):
                pltpu.sync_copy(row2.at[k * MROW_S + j], ibs[j])
            cps = [pltpu.async_copy(ob, acc1.at[ibs[j]], sem, add=True)
                   for j in range(MROW_S)]
            for h_ in cps:
                h_.wait()

        return carry

    lax.fori_loop(0, IT_C, body, 0)
    plsc.subcore_barrier()

    @pl.when(c == 0)
    def _():
        pltpu.sync_copy(acc1.at[pl.ds(s * STR, STR)], cnt0.at[pl.ds(s * STR, STR)])

    @pl.when(c == 1)
    def _():
        pltpu.sync_copy(acc1.at[pl.ds(s * STR, STR)], cnt1.at[pl.ds(s * STR, STR)])


_count_call = pl.kernel(
    _count_body,
    out_type=(
        jax.ShapeDtypeStruct((NNP,), jnp.float32),
        jax.ShapeDtypeStruct((NNP,), jnp.float32),
    ),
    mesh=_SC_MESH,
    compiler_params=pltpu.CompilerParams(use_tc_tiling_on_sc=False),
    scratch_types=(
        *([pltpu.VMEM((G,), jnp.int32)] * MROW_S),
        pltpu.VMEM((G,), jnp.float32),
        pltpu.VMEM_SHARED((NNP,), jnp.float32),
        pltpu.SemaphoreType.DMA,
    ),
)


def _silu(v):
    return v * jax.nn.sigmoid(v)


def _dot(a, b):
    return jnp.dot(a, b, preferred_element_type=jnp.float32)


def _edge_body(hrow, hcol, cr, cc, ea,
               w0h1, w0h2, w0e, w0r, b0, w1, b1, wc0, bc0, wc1t,
               m2a_o, m2b_o, tx_o, ty_o, tz_o):
    dx = cr[:, 0:1] - cc[:, 0:1]
    dy = cr[:, 1:2] - cc[:, 1:2]
    dz = cr[:, 2:3] - cc[:, 2:3]
    rad = dx * dx + dy * dy + dz * dz
    t = (_dot(hrow[...], w0h1[...]) + _dot(hcol[...], w0h2[...])
         + _dot(ea[...], w0e[...]) + rad * w0r[...] + b0[...])
    m = _silu(t)
    m = _silu(_dot(m, w1[...]) + b1[...])
    p = _silu(_dot(m, wc0[...]) + bc0[...])
    wgt = jnp.sum(p * wc1t[...], axis=1, keepdims=True)
    m2a_o[...] = m[:, :HHALF]
    m2b_o[...] = m[:, HHALF:]
    tx_o[...] = jnp.reshape(dx * wgt, (BE,))
    ty_o[...] = jnp.reshape(dy * wgt, (BE,))
    tz_o[...] = jnp.reshape(dz * wgt, (BE,))


def _wspec(shape):
    nd = len(shape)
    return pl.BlockSpec(shape, lambda i: (0,) * nd)


_edge_call = pl.pallas_call(
    _edge_body,
    grid=(NEP // BE,),
    in_specs=[
        pl.BlockSpec((BE, HD), lambda i: (i, 0)),
        pl.BlockSpec((BE, HD), lambda i: (i, 0)),
        pl.BlockSpec((BE, CW), lambda i: (i, 0)),
        pl.BlockSpec((BE, CW), lambda i: (i, 0)),
        pl.BlockSpec((BE, ED), lambda i: (i, 0)),
        _wspec((HD, HD)), _wspec((HD, HD)), _wspec((ED, HD)),
        _wspec((1, HD)), _wspec((1, HD)),
        _wspec((HD, HD)), _wspec((1, HD)),
        _wspec((HD, HD)), _wspec((1, HD)), _wspec((1, HD)),
    ],
    out_specs=[
        pl.BlockSpec((BE, HHALF), lambda i: (i, 0)),
        pl.BlockSpec((BE, HHALF), lambda i: (i, 0)),
        pl.BlockSpec((BE,), lambda i: (i,)),
        pl.BlockSpec((BE,), lambda i: (i,)),
        pl.BlockSpec((BE,), lambda i: (i,)),
    ],
    out_shape=(
        jax.ShapeDtypeStruct((NEP, HHALF), jnp.float32),
        jax.ShapeDtypeStruct((NEP, HHALF), jnp.float32),
        jax.ShapeDtypeStruct((NEP,), jnp.float32),
        jax.ShapeDtypeStruct((NEP,), jnp.float32),
        jax.ShapeDtypeStruct((NEP,), jnp.float32),
    ),
)


def _node_body(last, hh, aL, aR, ax, ay, az, c0, c1, cx, cy, cz,
               wn0h, wn0a, wn0b, bn0, wn1, bn1, wo, bo,
               h_o, cx_o, cy_o, cz_o):
    hhv = hh[...]
    t = (_dot(hhv, wn0h[...]) + _dot(aL[...], wn0a[...])
         + _dot(aR[...], wn0b[...]) + bn0[...])
    o = _dot(_silu(t), wn1[...]) + bn1[...]
    hn = hhv + o
    if last:
        h_o[...] = _dot(hn, wo[...]) + bo[...]
    else:
        h_o[...] = hn
    cnt = jnp.maximum(c0[...] + c1[...], 1.0)
    inv = 1.0 / cnt
    cx_o[...] = cx[...] + ax[...] * inv
    cy_o[...] = cy[...] + ay[...] * inv
    cz_o[...] = cz[...] + az[...] * inv


def _make_node_call(last):
    return pl.pallas_call(
        functools.partial(_node_body, last),
        grid=(NNP // BN,),
        in_specs=[
            pl.BlockSpec((BN, HD), lambda i: (i, 0)),
            pl.BlockSpec((BN, HHALF), lambda i: (i, 0)),
            pl.BlockSpec((BN, HHALF), lambda i: (i, 0)),
            pl.BlockSpec((BN,), lambda i: (i,)),
            pl.BlockSpec((BN,), lambda i: (i,)),
            pl.BlockSpec((BN,), lambda i: (i,)),
            pl.BlockSpec((BN,), lambda i: (i,)),
            pl.BlockSpec((BN,), lambda i: (i,)),
            pl.BlockSpec((BN,), lambda i: (i,)),
            pl.BlockSpec((BN,), lambda i: (i,)),
            pl.BlockSpec((BN,), lambda i: (i,)),
            _wspec((HD, HD)), _wspec((HHALF, HD)), _wspec((HHALF, HD)),
            _wspec((1, HD)),
            _wspec((HD, HD)), _wspec((1, HD)),
            _wspec((HD, HD)), _wspec((1, HD)),
        ],
        out_specs=[
            pl.BlockSpec((BN, HD), lambda i: (i, 0)),
            pl.BlockSpec((BN,), lambda i: (i,)),
            pl.BlockSpec((BN,), lambda i: (i,)),
            pl.BlockSpec((BN,), lambda i: (i,)),
        ],
        out_shape=(
            jax.ShapeDtypeStruct((NNP, HD), jnp.float32),
            jax.ShapeDtypeStruct((NNP,), jnp.float32),
            jax.ShapeDtypeStruct((NNP,), jnp.float32),
            jax.ShapeDtypeStruct((NNP,), jnp.float32),
        ),
    )


_node_call = _make_node_call(False)
_node_call_last = _make_node_call(True)


def _embin_body(h, wi, bi, o):
    o[...] = _dot(h[...], wi[...]) + bi[...]


_embin_call = pl.pallas_call(
    _embin_body,
    grid=(NNP // BN,),
    in_specs=[
        pl.BlockSpec((BN, HD), lambda i: (i, 0)),
        _wspec((HD, HD)), _wspec((1, HD)),
    ],
    out_specs=pl.BlockSpec((BN, HD), lambda i: (i, 0)),
    out_shape=jax.ShapeDtypeStruct((NNP, HD), jnp.float32),
)


def kernel(h, x, edge_index, edge_attr, params):
    row2 = edge_index[0].reshape(NE // G, G)
    col2 = edge_index[1].reshape(NE // G, G)
    z16 = jnp.zeros((SACC, HHALF), jnp.float32)
    z1 = jnp.zeros((STR,), jnp.float32)
    ones = jnp.ones((G,), jnp.float32)

    cnt0, cnt1 = _count_call(row2, ones, z1)

    hp = jnp.pad(h, ((0, NNP - NN), (0, 0)))
    xp = jnp.pad(x, ((0, NNP - NN), (0, 0)))
    eap = jnp.pad(edge_attr, ((0, NEP - NE), (0, 0)))
    hh = _embin_call(hp, params["emb_in"]["W"], params["emb_in"]["b"][None, :])
    cx = xp[:, 0]
    cy = xp[:, 1]
    cz = xp[:, 2]
    pad = jnp.zeros((NNP, CW - 3), jnp.float32)

    n_layers = len(params["layers"])
    for l, L in enumerate(params["layers"]):
        cpt = jnp.concatenate(
            [cx[:, None], cy[:, None], cz[:, None], pad], axis=1)
        hrow, hcol, crow, ccol = _gather_call(hh, cpt, row2, col2)

        W0 = L["edge_mlp_0"]["W"]
        m2a, m2b, tx, ty, tz = _edge_call(
            hrow, hcol, crow, ccol, eap,
            W0[0:HD], W0[HD:2 * HD], W0[2 * HD + 1:], W0[2 * HD:2 * HD + 1],
            L["edge_mlp_0"]["b"][None, :],
            L["edge_mlp_1"]["W"], L["edge_mlp_1"]["b"][None, :],
            L["coord_mlp_0"]["W"], L["coord_mlp_0"]["b"][None, :],
            L["coord_mlp_1"]["W"].T,
        )

        aggA, aggB, ax, ay, az = _scatter_call(
            m2a, m2b, tx, ty, tz, row2, z16, z1)

        Wn0 = L["node_mlp_0"]["W"]
        call = _node_call_last if l == n_layers - 1 else _node_call
        hh, cx, cy, cz = call(
            hh, aggA, aggB, ax, ay, az, cnt0, cnt1, cx, cy, cz,
            Wn0[0:HD], Wn0[HD:HD + HHALF], Wn0[HD + HHALF:], L["node_mlp_0"]["b"][None, :],
            L["node_mlp_1"]["W"], L["node_mlp_1"]["b"][None, :],
            params["emb_out"]["W"], params["emb_out"]["b"][None, :],
        )

    pts = jnp.concatenate(
        [cx[:NN, None], cy[:NN, None], cz[:NN, None]], axis=1)
    return (hh[:NN], pts)
